# baseline (speedup 1.0000x reference)
import functools

import jax
import jax.numpy as jnp
from jax import lax
from jax.experimental import pallas as pl
from jax.experimental.pallas import tpu as pltpu
from jax.experimental.pallas import tpu_sc as plsc

DM = 4096
NF = 32768
BT = 32
KTOP = 32
L = 16
NC, NS = 2, 16
NV = NF // L
GRP = 8
GROWS = 8

BN = 512


def _enc_body(x_ref, bdec_ref, w_ref, benc_ref, o_ref):
    xm = x_ref[...] - bdec_ref[...]
    o_ref[...] = (
        jnp.dot(xm, w_ref[...], preferred_element_type=jnp.float32)
        + benc_ref[...]
    )


def _encode(x, W_enc, b_enc, b_dec):
    return pl.pallas_call(
        _enc_body,
        grid=(NF // BN,),
        in_specs=[
            pl.BlockSpec((BT, DM), lambda i: (0, 0)),
            pl.BlockSpec((1, DM), lambda i: (0, 0)),
            pl.BlockSpec((DM, BN), lambda i: (0, i)),
            pl.BlockSpec((1, BN), lambda i: (0, i)),
        ],
        out_specs=pl.BlockSpec((BT, BN), lambda i: (0, i)),
        out_shape=jax.ShapeDtypeStruct((BT, NF), jnp.float32),
    )(x, b_dec.reshape(1, DM), W_enc, b_enc.reshape(1, NF))


def _merge16(hik, hii, lok, loi, sk, si):
    rk = lax.rev(sk, (0,))
    ri = lax.rev(si, (0,))
    p = lok >= rk
    ak = jnp.where(p, lok, rk)
    ai = jnp.where(p, loi, ri)
    ak, ai = plsc.sort_key_val(ak, ai, descending=True)
    rk = lax.rev(ak, (0,))
    ri = lax.rev(ai, (0,))
    p = hik >= rk
    nk = jnp.where(p, hik, rk)
    ni = jnp.where(p, hii, ri)
    mk = jnp.where(p, rk, hik)
    mi = jnp.where(p, ri, hii)
    nk, ni = plsc.sort_key_val(nk, ni, descending=True)
    mk, mi = plsc.sort_key_val(mk, mi, descending=True)
    return nk, ni, mk, mi


def _sc_body(pre_hbm, wdec_hbm, bdec_hbm, f_hbm, recon_hbm,
             row_v, bufa_v, bufb_v, acc_v, bdec_v, idx_v,
             semg_a, semg_b, semb, semf):
    wid = lax.axis_index("s") * NC + lax.axis_index("c")
    cpb = pltpu.async_copy(bdec_hbm, bdec_v, semb)
    pltpu.sync_copy(pre_hbm.at[wid], row_v)

    neg = jnp.float32(-3.0e38)
    lane = lax.iota(jnp.int32, L)

    def scan_group(g, carry):
        hik, hii, lok, loi, thr = carry
        base = g * (GRP * L)
        vs = [row_v[pl.ds(base + u * L, L)] for u in range(GRP)]
        m = vs[0]
        for u in range(1, GRP):
            m = jnp.maximum(m, vs[u])
        gmax = lax.reduce_max(m, (0,))

        def hit(c):
            def one(c, v, off):
                def do(c):
                    hik, hii, lok, loi, _ = c
                    sk, si = plsc.sort_key_val(v, lane + off,
                                               descending=True)
                    hik, hii, lok, loi = _merge16(hik, hii, lok, loi, sk, si)
                    return hik, hii, lok, loi, lax.reduce_min(lok, (0,))

                vmax = lax.reduce_max(v, (0,))
                return lax.cond(vmax > c[4], do, lambda c: c, c)

            for u in range(GRP):
                c = one(c, vs[u], base + u * L)
            return c

        return lax.cond(gmax > thr, hit, lambda c: c, carry)

    init = (jnp.full((L,), neg), jnp.zeros((L,), jnp.int32),
            jnp.full((L,), neg), jnp.zeros((L,), jnp.int32),
            neg)
    hik, hii, lok, loi, _ = lax.fori_loop(0, NV // GRP, scan_group, init)

    idx_v[pl.ds(0, L)] = hii
    idx_v[pl.ds(L, L)] = loi
    gathers = [
        pltpu.async_copy(
            wdec_hbm.at[idx_v.at[pl.ds(k * GROWS, GROWS)]],
            bufa_v if k % 2 == 0 else bufb_v,
            semg_a if k % 2 == 0 else semg_b)
        for k in range(1)
    ]

    zero = jnp.zeros((L,), jnp.float32)

    def zbody(i, _):
        base = i * (8 * L)
        for u in range(8):
            row_v[pl.ds(base + u * L, L)] = zero
        return 0

    lax.fori_loop(0, NV // 8, zbody, 0)
    plsc.store_scatter(row_v, [hii], jnp.maximum(hik, 0.0))
    plsc.store_scatter(row_v, [loi], jnp.maximum(lok, 0.0))
    cpf = pltpu.async_copy(row_v, f_hbm.at[wid], semf)

    vh = jnp.maximum(hik, 0.0)
    vl = jnp.maximum(lok, 0.0)
    ws_all = [vh[r] for r in range(L)] + [vl[r] for r in range(L)]
    cpb.wait()

    nchunks = (2 * L) // GROWS
    for k in range(nchunks):
        if k + 1 < nchunks:
            gathers.append(pltpu.async_copy(
                wdec_hbm.at[idx_v.at[pl.ds((k + 1) * GROWS, GROWS)]],
                bufb_v if k % 2 == 0 else bufa_v,
                semg_b if k % 2 == 0 else semg_a))
        gathers[k].wait()
        buf = bufa_v if k % 2 == 0 else bufb_v
        ws = ws_all[k * GROWS:(k + 1) * GROWS]
        src = bdec_v if k == 0 else acc_v

        def jbody(j, _, buf=buf, ws=ws, src=src):
            o = j * L
            a = src[pl.ds(o, L)]
            for r in range(GROWS):
                a = a + ws[r] * buf[r, pl.ds(o, L)]
            acc_v[pl.ds(o, L)] = a
            return 0

        lax.fori_loop(0, DM // L, jbody, 0)

    cpf.wait()
    pltpu.sync_copy(acc_v, recon_hbm.at[wid])


def _decode_topk(pre, W_dec, b_dec):
    mesh = plsc.VectorSubcoreMesh(
        core_axis_name="c", subcore_axis_name="s",
        num_cores=NC, num_subcores=NS)
    fn = functools.partial(
        pl.kernel,
        out_type=(jax.ShapeDtypeStruct((BT, NF), jnp.float32),
                  jax.ShapeDtypeStruct((BT, DM), jnp.float32)),
        mesh=mesh,
        scratch_types=[
            pltpu.VMEM((NF,), jnp.float32),
            pltpu.VMEM((GROWS, DM), jnp.float32),
            pltpu.VMEM((GROWS, DM), jnp.float32),
            pltpu.VMEM((DM,), jnp.float32),
            pltpu.VMEM((DM,), jnp.float32),
            pltpu.VMEM((2 * L,), jnp.int32),
            pltpu.SemaphoreType.DMA,
            pltpu.SemaphoreType.DMA,
            pltpu.SemaphoreType.DMA,
            pltpu.SemaphoreType.DMA,
        ],
        compiler_params=pltpu.CompilerParams(needs_layout_passes=False),
    )(_sc_body)
    return fn(pre, W_dec, b_dec)


def kernel(x, W_enc, b_enc, W_dec, b_dec):
    pre = _encode(x, W_enc, b_enc, b_dec)
    f, recon = _decode_topk(pre, W_dec, b_dec)
    return (recon, f)


# speedup vs baseline: 1.0270x; 1.0270x over previous
import functools

import jax
import jax.numpy as jnp
from jax import lax
from jax.experimental import pallas as pl
from jax.experimental.pallas import tpu as pltpu
from jax.experimental.pallas import tpu_sc as plsc

DM = 4096
NF = 32768
BT = 32
KTOP = 32
L = 16
NC, NS = 2, 16
NV = NF // L
GRP = 8
GROWS = 8

BN = 512


def _enc_body(x_ref, bdec_ref, w_ref, benc_ref, o_ref):
    xm = x_ref[...] - bdec_ref[...]
    o_ref[...] = (
        jnp.dot(xm, w_ref[...], preferred_element_type=jnp.float32)
        + benc_ref[...]
    )


def _encode(x, W_enc, b_enc, b_dec):
    return pl.pallas_call(
        _enc_body,
        grid=(NF // BN,),
        in_specs=[
            pl.BlockSpec((BT, DM), lambda i: (0, 0)),
            pl.BlockSpec((1, DM), lambda i: (0, 0)),
            pl.BlockSpec((DM, BN), lambda i: (0, i)),
            pl.BlockSpec((1, BN), lambda i: (0, i)),
        ],
        out_specs=pl.BlockSpec((BT, BN), lambda i: (0, i)),
        out_shape=jax.ShapeDtypeStruct((BT, NF), jnp.float32),
    )(x, b_dec.reshape(1, DM), W_enc, b_enc.reshape(1, NF))


def _merge16(hik, hii, lok, loi, sk, si):
    rk = lax.rev(sk, (0,))
    ri = lax.rev(si, (0,))
    p = lok >= rk
    ak = jnp.where(p, lok, rk)
    ai = jnp.where(p, loi, ri)
    ak, ai = plsc.sort_key_val(ak, ai, descending=True)
    rk = lax.rev(ak, (0,))
    ri = lax.rev(ai, (0,))
    p = hik >= rk
    nk = jnp.where(p, hik, rk)
    ni = jnp.where(p, hii, ri)
    mk = jnp.where(p, rk, hik)
    mi = jnp.where(p, ri, hii)
    nk, ni = plsc.sort_key_val(nk, ni, descending=True)
    mk, mi = plsc.sort_key_val(mk, mi, descending=True)
    return nk, ni, mk, mi


def _sc_body(pre_hbm, wdec_hbm, bdec_hbm, f_hbm, recon_hbm,
             row_v, bufa_v, bufb_v, acc_v, bdec_v, idx_v,
             semg_a, semg_b, semb, semf):
    wid = lax.axis_index("s") * NC + lax.axis_index("c")
    cpb = pltpu.async_copy(bdec_hbm, bdec_v, semb)
    pltpu.sync_copy(pre_hbm.at[wid], row_v)

    neg = jnp.float32(-3.0e38)
    lane = lax.iota(jnp.int32, L)

    def boot(i, c):
        ca, cb = c
        base = i * (8 * L)
        for u in range(0, 8, 2):
            ca = jnp.maximum(ca, row_v[pl.ds(base + u * L, L)])
            cb = jnp.maximum(cb, row_v[pl.ds(base + (u + 1) * L, L)])
        return ca, cb

    ca, cb = lax.fori_loop(0, NV // 8, boot,
                           (jnp.full((L,), neg), jnp.full((L,), neg)))
    t0 = lax.reduce_min(jnp.minimum(ca, cb), (0,))

    SG = 32

    def scan_group(g, carry):
        hik, hii, lok, loi, thr = carry
        base = g * (SG * L)
        vs = [row_v[pl.ds(base + u * L, L)] for u in range(SG)]
        m = vs[0]
        for u in range(1, SG):
            m = jnp.maximum(m, vs[u])
        pc = plsc.all_reduce_population_count(m >= thr)

        def one(c, v, off):
            def do(c):
                hik, hii, lok, loi, _ = c
                sk, si = plsc.sort_key_val(v, lane + off, descending=True)
                hik, hii, lok, loi = _merge16(hik, hii, lok, loi, sk, si)
                thr = jnp.maximum(t0, lax.reduce_min(lok, (0,)))
                return hik, hii, lok, loi, thr

            vmax = lax.reduce_max(v, (0,))
            return lax.cond(vmax >= c[4], do, lambda c: c, c)

        def hit(c):
            for s0 in range(0, SG, GRP):
                sub = vs[s0:s0 + GRP]
                sm = sub[0]
                for u in range(1, GRP):
                    sm = jnp.maximum(sm, sub[u])
                smax = lax.reduce_max(sm, (0,))

                def subhit(c, sub=sub, s0=s0):
                    for u in range(GRP):
                        c = one(c, sub[u], base + (s0 + u) * L)
                    return c

                c = lax.cond(smax >= c[4], subhit, lambda c: c, c)
            return c

        return lax.cond(pc[0] > 0, hit, lambda c: c, carry)

    init = (jnp.full((L,), neg), jnp.zeros((L,), jnp.int32),
            jnp.full((L,), neg), jnp.zeros((L,), jnp.int32),
            t0)
    hik, hii, lok, loi, _ = lax.fori_loop(0, NV // SG, scan_group, init)

    idx_v[pl.ds(0, L)] = hii
    idx_v[pl.ds(L, L)] = loi
    gathers = [
        pltpu.async_copy(
            wdec_hbm.at[idx_v.at[pl.ds(k * GROWS, GROWS)]],
            bufa_v if k % 2 == 0 else bufb_v,
            semg_a if k % 2 == 0 else semg_b)
        for k in range(1)
    ]

    zero = jnp.zeros((L,), jnp.float32)

    def zbody(i, _):
        base = i * (8 * L)
        for u in range(8):
            row_v[pl.ds(base + u * L, L)] = zero
        return 0

    lax.fori_loop(0, NV // 8, zbody, 0)
    plsc.store_scatter(row_v, [hii], jnp.maximum(hik, 0.0))
    plsc.store_scatter(row_v, [loi], jnp.maximum(lok, 0.0))
    cpf = pltpu.async_copy(row_v, f_hbm.at[wid], semf)

    vh = jnp.maximum(hik, 0.0)
    vl = jnp.maximum(lok, 0.0)
    ws_all = [vh[r] for r in range(L)] + [vl[r] for r in range(L)]
    cpb.wait()

    nchunks = (2 * L) // GROWS
    for k in range(nchunks):
        if k + 1 < nchunks:
            gathers.append(pltpu.async_copy(
                wdec_hbm.at[idx_v.at[pl.ds((k + 1) * GROWS, GROWS)]],
                bufb_v if k % 2 == 0 else bufa_v,
                semg_b if k % 2 == 0 else semg_a))
        gathers[k].wait()
        buf = bufa_v if k % 2 == 0 else bufb_v
        ws = ws_all[k * GROWS:(k + 1) * GROWS]
        src = bdec_v if k == 0 else acc_v

        def jbody(j, _, buf=buf, ws=ws, src=src):
            o = j * L
            a = src[pl.ds(o, L)]
            for r in range(GROWS):
                a = a + ws[r] * buf[r, pl.ds(o, L)]
            acc_v[pl.ds(o, L)] = a
            return 0

        lax.fori_loop(0, DM // L, jbody, 0)

    cpf.wait()
    pltpu.sync_copy(acc_v, recon_hbm.at[wid])


def _decode_topk(pre, W_dec, b_dec):
    mesh = plsc.VectorSubcoreMesh(
        core_axis_name="c", subcore_axis_name="s",
        num_cores=NC, num_subcores=NS)
    fn = functools.partial(
        pl.kernel,
        out_type=(jax.ShapeDtypeStruct((BT, NF), jnp.float32),
                  jax.ShapeDtypeStruct((BT, DM), jnp.float32)),
        mesh=mesh,
        scratch_types=[
            pltpu.VMEM((NF,), jnp.float32),
            pltpu.VMEM((GROWS, DM), jnp.float32),
            pltpu.VMEM((GROWS, DM), jnp.float32),
            pltpu.VMEM((DM,), jnp.float32),
            pltpu.VMEM((DM,), jnp.float32),
            pltpu.VMEM((2 * L,), jnp.int32),
            pltpu.SemaphoreType.DMA,
            pltpu.SemaphoreType.DMA,
            pltpu.SemaphoreType.DMA,
            pltpu.SemaphoreType.DMA,
        ],
        compiler_params=pltpu.CompilerParams(needs_layout_passes=False),
    )(_sc_body)
    return fn(pre, W_dec, b_dec)


def kernel(x, W_enc, b_enc, W_dec, b_dec):
    pre = _encode(x, W_enc, b_enc, b_dec)
    f, recon = _decode_topk(pre, W_dec, b_dec)
    return (recon, f)


# speedup vs baseline: 1.0276x; 1.0005x over previous
import functools

import jax
import jax.numpy as jnp
from jax import lax
from jax.experimental import pallas as pl
from jax.experimental.pallas import tpu as pltpu
from jax.experimental.pallas import tpu_sc as plsc

DM = 4096
NF = 32768
BT = 32
KTOP = 32
L = 16
NC, NS = 2, 16
NV = NF // L
GRP = 8
GROWS = 8

BN = 512


def _enc_body(x_ref, bdec_ref, w_ref, benc_ref, o_ref):
    xm = x_ref[...] - bdec_ref[...]
    o_ref[...] = (
        jnp.dot(xm, w_ref[...], preferred_element_type=jnp.float32)
        + benc_ref[...]
    )


def _encode(x, W_enc, b_enc, b_dec):
    return pl.pallas_call(
        _enc_body,
        grid=(NF // BN,),
        in_specs=[
            pl.BlockSpec((BT, DM), lambda i: (0, 0)),
            pl.BlockSpec((1, DM), lambda i: (0, 0)),
            pl.BlockSpec((DM, BN), lambda i: (0, i)),
            pl.BlockSpec((1, BN), lambda i: (0, i)),
        ],
        out_specs=pl.BlockSpec((BT, BN), lambda i: (0, i)),
        out_shape=jax.ShapeDtypeStruct((BT, NF), jnp.float32),
    )(x, b_dec.reshape(1, DM), W_enc, b_enc.reshape(1, NF))


def _merge16(hik, hii, lok, loi, sk, si):
    rk = lax.rev(sk, (0,))
    ri = lax.rev(si, (0,))
    p = lok >= rk
    ak = jnp.where(p, lok, rk)
    ai = jnp.where(p, loi, ri)
    ak, ai = plsc.sort_key_val(ak, ai, descending=True)
    rk = lax.rev(ak, (0,))
    ri = lax.rev(ai, (0,))
    p = hik >= rk
    nk = jnp.where(p, hik, rk)
    ni = jnp.where(p, hii, ri)
    mk = jnp.where(p, rk, hik)
    mi = jnp.where(p, ri, hii)
    nk, ni = plsc.sort_key_val(nk, ni, descending=True)
    mk, mi = plsc.sort_key_val(mk, mi, descending=True)
    return nk, ni, mk, mi


def _sc_body(pre_hbm, wdec_hbm, bdec_hbm, f_hbm, recon_hbm,
             row_v, bufa_v, bufb_v, acc_v, bdec_v, idx_v,
             semg_a, semg_b, semb, semf):
    wid = lax.axis_index("s") * NC + lax.axis_index("c")
    cpb = pltpu.async_copy(bdec_hbm, bdec_v, semb)
    with jax.named_scope("rowdma"):
        pltpu.sync_copy(pre_hbm.at[wid], row_v)

    neg = jnp.float32(-3.0e38)
    lane = lax.iota(jnp.int32, L)

    def boot(i, c):
        ca, cb = c
        base = i * (8 * L)
        for u in range(0, 8, 2):
            ca = jnp.maximum(ca, row_v[pl.ds(base + u * L, L)])
            cb = jnp.maximum(cb, row_v[pl.ds(base + (u + 1) * L, L)])
        return ca, cb

    with jax.named_scope("boot"):
        ca, cb = lax.fori_loop(0, NV // 8, boot,
                               (jnp.full((L,), neg), jnp.full((L,), neg)))
        t0 = lax.reduce_min(jnp.minimum(ca, cb), (0,))

    SG = 32

    def scan_group(g, carry):
        hik, hii, lok, loi, thr = carry
        base = g * (SG * L)
        vs = [row_v[pl.ds(base + u * L, L)] for u in range(SG)]
        m = vs[0]
        for u in range(1, SG):
            m = jnp.maximum(m, vs[u])
        pc = plsc.all_reduce_population_count(m >= thr)

        def one(c, v, off):
            def do(c):
                hik, hii, lok, loi, _ = c
                sk, si = plsc.sort_key_val(v, lane + off, descending=True)
                hik, hii, lok, loi = _merge16(hik, hii, lok, loi, sk, si)
                thr = jnp.maximum(t0, lax.reduce_min(lok, (0,)))
                return hik, hii, lok, loi, thr

            vmax = lax.reduce_max(v, (0,))
            return lax.cond(vmax >= c[4], do, lambda c: c, c)

        def hit(c):
            for s0 in range(0, SG, GRP):
                sub = vs[s0:s0 + GRP]
                sm = sub[0]
                for u in range(1, GRP):
                    sm = jnp.maximum(sm, sub[u])
                smax = lax.reduce_max(sm, (0,))

                def subhit(c, sub=sub, s0=s0):
                    for u in range(GRP):
                        c = one(c, sub[u], base + (s0 + u) * L)
                    return c

                c = lax.cond(smax >= c[4], subhit, lambda c: c, c)
            return c

        return lax.cond(pc[0] > 0, hit, lambda c: c, carry)

    init = (jnp.full((L,), neg), jnp.zeros((L,), jnp.int32),
            jnp.full((L,), neg), jnp.zeros((L,), jnp.int32),
            t0)
    with jax.named_scope("scan"):
        hik, hii, lok, loi, _ = lax.fori_loop(0, NV // SG, scan_group, init)

    idx_v[pl.ds(0, L)] = hii
    idx_v[pl.ds(L, L)] = loi
    gathers = [
        pltpu.async_copy(
            wdec_hbm.at[idx_v.at[pl.ds(k * GROWS, GROWS)]],
            bufa_v if k % 2 == 0 else bufb_v,
            semg_a if k % 2 == 0 else semg_b)
        for k in range(1)
    ]

    zero = jnp.zeros((L,), jnp.float32)

    def zbody(i, _):
        base = i * (8 * L)
        for u in range(8):
            row_v[pl.ds(base + u * L, L)] = zero
        return 0

    with jax.named_scope("fbuild"):
        lax.fori_loop(0, NV // 8, zbody, 0)
        plsc.store_scatter(row_v, [hii], jnp.maximum(hik, 0.0))
        plsc.store_scatter(row_v, [loi], jnp.maximum(lok, 0.0))
        cpf = pltpu.async_copy(row_v, f_hbm.at[wid], semf)

    vh = jnp.maximum(hik, 0.0)
    vl = jnp.maximum(lok, 0.0)
    ws_all = [vh[r] for r in range(L)] + [vl[r] for r in range(L)]
    cpb.wait()

    nchunks = (2 * L) // GROWS
    scope = jax.named_scope("decode")
    scope.__enter__()
    for k in range(nchunks):
        if k + 1 < nchunks:
            gathers.append(pltpu.async_copy(
                wdec_hbm.at[idx_v.at[pl.ds((k + 1) * GROWS, GROWS)]],
                bufb_v if k % 2 == 0 else bufa_v,
                semg_b if k % 2 == 0 else semg_a))
        gathers[k].wait()
        buf = bufa_v if k % 2 == 0 else bufb_v
        ws = ws_all[k * GROWS:(k + 1) * GROWS]
        src = bdec_v if k == 0 else acc_v

        def jbody(j, _, buf=buf, ws=ws, src=src):
            o = j * L
            a = src[pl.ds(o, L)]
            for r in range(GROWS):
                a = a + ws[r] * buf[r, pl.ds(o, L)]
            acc_v[pl.ds(o, L)] = a
            return 0

        lax.fori_loop(0, DM // L, jbody, 0)

    cpf.wait()
    pltpu.sync_copy(acc_v, recon_hbm.at[wid])
    scope.__exit__(None, None, None)


def _decode_topk(pre, W_dec, b_dec):
    mesh = plsc.VectorSubcoreMesh(
        core_axis_name="c", subcore_axis_name="s",
        num_cores=NC, num_subcores=NS)
    fn = functools.partial(
        pl.kernel,
        out_type=(jax.ShapeDtypeStruct((BT, NF), jnp.float32),
                  jax.ShapeDtypeStruct((BT, DM), jnp.float32)),
        mesh=mesh,
        scratch_types=[
            pltpu.VMEM((NF,), jnp.float32),
            pltpu.VMEM((GROWS, DM), jnp.float32),
            pltpu.VMEM((GROWS, DM), jnp.float32),
            pltpu.VMEM((DM,), jnp.float32),
            pltpu.VMEM((DM,), jnp.float32),
            pltpu.VMEM((2 * L,), jnp.int32),
            pltpu.SemaphoreType.DMA,
            pltpu.SemaphoreType.DMA,
            pltpu.SemaphoreType.DMA,
            pltpu.SemaphoreType.DMA,
        ],
        compiler_params=pltpu.CompilerParams(needs_layout_passes=False),
    )(_sc_body)
    return fn(pre, W_dec, b_dec)


def kernel(x, W_enc, b_enc, W_dec, b_dec):
    pre = _encode(x, W_enc, b_enc, b_dec)
    f, recon = _decode_topk(pre, W_dec, b_dec)
    return (recon, f)


# speedup vs baseline: 1.0803x; 1.0513x over previous
import functools

import jax
import jax.numpy as jnp
from jax import lax
from jax.experimental import pallas as pl
from jax.experimental.pallas import tpu as pltpu
from jax.experimental.pallas import tpu_sc as plsc

DM = 4096
NF = 32768
BT = 32
KTOP = 32
L = 16
NC, NS = 2, 16
NV = NF // L
GRP = 8
GROWS = 8
CANDBUF = 2048

BN = 512


def _enc_body(x_ref, bdec_ref, w_ref, benc_ref, o_ref):
    xm = x_ref[...] - bdec_ref[...]
    o_ref[...] = (
        jnp.dot(xm, w_ref[...], preferred_element_type=jnp.float32)
        + benc_ref[...]
    )


def _encode(x, W_enc, b_enc, b_dec):
    return pl.pallas_call(
        _enc_body,
        grid=(NF // BN,),
        in_specs=[
            pl.BlockSpec((BT, DM), lambda i: (0, 0)),
            pl.BlockSpec((1, DM), lambda i: (0, 0)),
            pl.BlockSpec((DM, BN), lambda i: (0, i)),
            pl.BlockSpec((1, BN), lambda i: (0, i)),
        ],
        out_specs=pl.BlockSpec((BT, BN), lambda i: (0, i)),
        out_shape=jax.ShapeDtypeStruct((BT, NF), jnp.float32),
    )(x, b_dec.reshape(1, DM), W_enc, b_enc.reshape(1, NF))


def _merge16(hik, hii, lok, loi, sk, si):
    rk = lax.rev(sk, (0,))
    ri = lax.rev(si, (0,))
    p = lok >= rk
    ak = jnp.where(p, lok, rk)
    ai = jnp.where(p, loi, ri)
    ak, ai = plsc.sort_key_val(ak, ai, descending=True)
    rk = lax.rev(ak, (0,))
    ri = lax.rev(ai, (0,))
    p = hik >= rk
    nk = jnp.where(p, hik, rk)
    ni = jnp.where(p, hii, ri)
    mk = jnp.where(p, rk, hik)
    mi = jnp.where(p, ri, hii)
    nk, ni = plsc.sort_key_val(nk, ni, descending=True)
    mk, mi = plsc.sort_key_val(mk, mi, descending=True)
    return nk, ni, mk, mi


def _sc_body(pre_hbm, wdec_hbm, bdec_hbm, f_hbm, recon_hbm,
             row_v, bufa_v, bufb_v, acc_v, bdec_v, idx_v, cand_v, candi_v,
             semg_a, semg_b, semb, semf):
    wid = lax.axis_index("s") * NC + lax.axis_index("c")
    cpb = pltpu.async_copy(bdec_hbm, bdec_v, semb)
    with jax.named_scope("rowdma"):
        pltpu.sync_copy(pre_hbm.at[wid], row_v)

    neg = jnp.float32(-3.0e38)
    lane = lax.iota(jnp.int32, L)

    def boot(i, c):
        ca, cb = c
        base = i * (8 * L)
        for u in range(0, 8, 2):
            ca = jnp.maximum(ca, row_v[pl.ds(base + u * L, L)])
            cb = jnp.maximum(cb, row_v[pl.ds(base + (u + 1) * L, L)])
        return ca, cb

    with jax.named_scope("boot"):
        ca, cb = lax.fori_loop(0, NV // 8, boot,
                               (jnp.full((L,), neg), jnp.full((L,), neg)))
        t0 = lax.reduce_min(jnp.minimum(ca, cb), (0,))

    t0v = jnp.full((L,), t0)
    SG = 16

    def drain(ptr, top):
        ones = lane >= 0
        plsc.store_compressed(cand_v.at[pl.ds(ptr, L)], jnp.full((L,), neg),
                              mask=ones)
        nb = (ptr + L - 1) // L

        def dbody(b, c):
            hik, hii, lok, loi, thr = c
            v = cand_v[pl.ds(b * L, L)]
            iv = candi_v[pl.ds(b * L, L)]

            def do(c):
                hik, hii, lok, loi, _ = c
                sk, si = plsc.sort_key_val(v, iv, descending=True)
                hik, hii, lok, loi = _merge16(hik, hii, lok, loi, sk, si)
                thr = jnp.maximum(t0, lax.reduce_min(lok, (0,)))
                return hik, hii, lok, loi, thr

            vmax = lax.reduce_max(v, (0,))
            return lax.cond(vmax >= c[4], do, lambda c: c, c)

        return lax.fori_loop(0, nb, dbody, top)

    def scan_group(g, carry):
        ptr, top = carry[0], carry[1:]
        base = g * (SG * L)
        for u in range(SG):
            off = base + u * L
            v = row_v[pl.ds(off, L)]
            m = v >= t0v
            plsc.store_compressed(cand_v.at[pl.ds(ptr, L)], v, mask=m)
            plsc.store_compressed(candi_v.at[pl.ds(ptr, L)], lane + off,
                                  mask=m)
            ptr = ptr + plsc.all_reduce_population_count(m)[0]

        def flush(c):
            top = drain(c[0], c[1:])
            return (jnp.int32(0),) + top

        return lax.cond(ptr >= CANDBUF, flush, lambda c: c, (ptr,) + top)

    init = (jnp.int32(0),
            jnp.full((L,), neg), jnp.zeros((L,), jnp.int32),
            jnp.full((L,), neg), jnp.zeros((L,), jnp.int32),
            t0)
    with jax.named_scope("scan"):
        out = lax.fori_loop(0, NV // SG, scan_group, init)
        hik, hii, lok, loi, _ = drain(out[0], out[1:])

    idx_v[pl.ds(0, L)] = hii
    idx_v[pl.ds(L, L)] = loi
    gathers = [
        pltpu.async_copy(
            wdec_hbm.at[idx_v.at[pl.ds(k * GROWS, GROWS)]],
            bufa_v if k % 2 == 0 else bufb_v,
            semg_a if k % 2 == 0 else semg_b)
        for k in range(1)
    ]

    zero = jnp.zeros((L,), jnp.float32)

    def zbody(i, _):
        base = i * (8 * L)
        for u in range(8):
            row_v[pl.ds(base + u * L, L)] = zero
        return 0

    with jax.named_scope("fbuild"):
        lax.fori_loop(0, NV // 8, zbody, 0)
        plsc.store_scatter(row_v, [hii], jnp.maximum(hik, 0.0))
        plsc.store_scatter(row_v, [loi], jnp.maximum(lok, 0.0))
        cpf = pltpu.async_copy(row_v, f_hbm.at[wid], semf)

    vh = jnp.maximum(hik, 0.0)
    vl = jnp.maximum(lok, 0.0)
    ws_all = [vh[r] for r in range(L)] + [vl[r] for r in range(L)]
    cpb.wait()

    nchunks = (2 * L) // GROWS
    scope = jax.named_scope("decode")
    scope.__enter__()
    for k in range(nchunks):
        if k + 1 < nchunks:
            gathers.append(pltpu.async_copy(
                wdec_hbm.at[idx_v.at[pl.ds((k + 1) * GROWS, GROWS)]],
                bufb_v if k % 2 == 0 else bufa_v,
                semg_b if k % 2 == 0 else semg_a))
        gathers[k].wait()
        buf = bufa_v if k % 2 == 0 else bufb_v
        ws = ws_all[k * GROWS:(k + 1) * GROWS]
        src = bdec_v if k == 0 else acc_v

        def jbody(j, _, buf=buf, ws=ws, src=src):
            for q in range(4):
                o = j * (4 * L) + q * L
                a = src[pl.ds(o, L)]
                for r in range(GROWS):
                    a = a + ws[r] * buf[r, pl.ds(o, L)]
                acc_v[pl.ds(o, L)] = a
            return 0

        lax.fori_loop(0, DM // (4 * L), jbody, 0)

    cpf.wait()
    pltpu.sync_copy(acc_v, recon_hbm.at[wid])
    scope.__exit__(None, None, None)


def _decode_topk(pre, W_dec, b_dec):
    mesh = plsc.VectorSubcoreMesh(
        core_axis_name="c", subcore_axis_name="s",
        num_cores=NC, num_subcores=NS)
    fn = functools.partial(
        pl.kernel,
        out_type=(jax.ShapeDtypeStruct((BT, NF), jnp.float32),
                  jax.ShapeDtypeStruct((BT, DM), jnp.float32)),
        mesh=mesh,
        scratch_types=[
            pltpu.VMEM((NF,), jnp.float32),
            pltpu.VMEM((GROWS, DM), jnp.float32),
            pltpu.VMEM((GROWS, DM), jnp.float32),
            pltpu.VMEM((DM,), jnp.float32),
            pltpu.VMEM((DM,), jnp.float32),
            pltpu.VMEM((2 * L,), jnp.int32),
            pltpu.VMEM((CANDBUF + 18 * L,), jnp.float32),
            pltpu.VMEM((CANDBUF + 18 * L,), jnp.int32),
            pltpu.SemaphoreType.DMA,
            pltpu.SemaphoreType.DMA,
            pltpu.SemaphoreType.DMA,
            pltpu.SemaphoreType.DMA,
        ],
        compiler_params=pltpu.CompilerParams(needs_layout_passes=False),
    )(_sc_body)
    return fn(pre, W_dec, b_dec)


def kernel(x, W_enc, b_enc, W_dec, b_dec):
    pre = _encode(x, W_enc, b_enc, b_dec)
    f, recon = _decode_topk(pre, W_dec, b_dec)
    return (recon, f)


# speedup vs baseline: 1.1047x; 1.0226x over previous
import functools

import jax
import jax.numpy as jnp
from jax import lax
from jax.experimental import pallas as pl
from jax.experimental.pallas import tpu as pltpu
from jax.experimental.pallas import tpu_sc as plsc

DM = 4096
NF = 32768
BT = 32
KTOP = 32
L = 16
NC, NS = 2, 16
NV = NF // L
GRP = 8
GROWS = 8
CANDBUF = 1024

BN = 512


def _enc_body(x_ref, bdec_ref, w_ref, benc_ref, o_ref):
    xm = x_ref[...] - bdec_ref[...]
    o_ref[...] = (
        jnp.dot(xm, w_ref[...], preferred_element_type=jnp.float32)
        + benc_ref[...]
    )


def _encode(x, W_enc, b_enc, b_dec):
    return pl.pallas_call(
        _enc_body,
        grid=(NF // BN,),
        in_specs=[
            pl.BlockSpec((BT, DM), lambda i: (0, 0)),
            pl.BlockSpec((1, DM), lambda i: (0, 0)),
            pl.BlockSpec((DM, BN), lambda i: (0, i)),
            pl.BlockSpec((1, BN), lambda i: (0, i)),
        ],
        out_specs=pl.BlockSpec((BT, BN), lambda i: (0, i)),
        out_shape=jax.ShapeDtypeStruct((BT, NF), jnp.float32),
    )(x, b_dec.reshape(1, DM), W_enc, b_enc.reshape(1, NF))


def _merge16(hik, hii, lok, loi, sk, si):
    rk = lax.rev(sk, (0,))
    ri = lax.rev(si, (0,))
    p = lok >= rk
    ak = jnp.where(p, lok, rk)
    ai = jnp.where(p, loi, ri)
    ak, ai = plsc.sort_key_val(ak, ai, descending=True)
    rk = lax.rev(ak, (0,))
    ri = lax.rev(ai, (0,))
    p = hik >= rk
    nk = jnp.where(p, hik, rk)
    ni = jnp.where(p, hii, ri)
    mk = jnp.where(p, rk, hik)
    mi = jnp.where(p, ri, hii)
    nk, ni = plsc.sort_key_val(nk, ni, descending=True)
    mk, mi = plsc.sort_key_val(mk, mi, descending=True)
    return nk, ni, mk, mi


def _sc_body(pre_hbm, wdec_hbm, bdec_hbm, f_hbm, recon_hbm,
             row_v, bufa_v, bufb_v, acc_v, bdec_v, idx_v,
             cand0_v, cand1_v, cand2_v, cand3_v,
             candi0_v, candi1_v, candi2_v, candi3_v,
             semg_a, semg_b, semb, semf):
    wid = lax.axis_index("s") * NC + lax.axis_index("c")
    cpb = pltpu.async_copy(bdec_hbm, bdec_v, semb)
    with jax.named_scope("rowdma"):
        pltpu.sync_copy(pre_hbm.at[wid], row_v)

    neg = jnp.float32(-3.0e38)
    lane = lax.iota(jnp.int32, L)

    def boot(i, c):
        ca, cb = c
        base = i * (8 * L)
        for u in range(0, 8, 2):
            ca = jnp.maximum(ca, row_v[pl.ds(base + u * L, L)])
            cb = jnp.maximum(cb, row_v[pl.ds(base + (u + 1) * L, L)])
        return ca, cb

    with jax.named_scope("boot"):
        ca, cb = lax.fori_loop(0, NV // 8, boot,
                               (jnp.full((L,), neg), jnp.full((L,), neg)))
        t0 = lax.reduce_min(jnp.minimum(ca, cb), (0,))

    t0v = jnp.full((L,), t0)
    SG = 16
    NCH = 4
    cands = [cand0_v, cand1_v, cand2_v, cand3_v]
    candis = [candi0_v, candi1_v, candi2_v, candi3_v]

    def drain(ptrs, top):
        ones = lane >= 0
        for c in range(NCH):
            plsc.store_compressed(cands[c].at[pl.ds(ptrs[c], L)],
                                  jnp.full((L,), neg), mask=ones)
        del ones

        def dbody(b, c, cv=None, civ=None):
            hik, hii, lok, loi, thr = c
            v = cv[pl.ds(b * L, L)]
            iv = civ[pl.ds(b * L, L)]

            def do(c):
                hik, hii, lok, loi, _ = c
                sk, si = plsc.sort_key_val(v, iv, descending=True)
                hik, hii, lok, loi = _merge16(hik, hii, lok, loi, sk, si)
                thr = jnp.maximum(t0, lax.reduce_min(lok, (0,)))
                return hik, hii, lok, loi, thr

            vmax = lax.reduce_max(v, (0,))
            return lax.cond(vmax >= c[4], do, lambda c: c, c)

        for c in range(NCH):
            nb = (ptrs[c] + L - 1) // L
            top = lax.fori_loop(
                0, nb,
                functools.partial(dbody, cv=cands[c], civ=candis[c]), top)
        return top

    def scan_group(g, carry):
        ptrs, top = list(carry[0]), carry[1:]
        base = g * (SG * L)
        for u in range(SG):
            c = u % NCH
            off = base + u * L
            v = row_v[pl.ds(off, L)]
            m = v >= t0v
            plsc.store_compressed(cands[c].at[pl.ds(ptrs[c], L)], v, mask=m)
            plsc.store_compressed(candis[c].at[pl.ds(ptrs[c], L)],
                                  lane + off, mask=m)
            ptrs[c] = ptrs[c] + plsc.all_reduce_population_count(m)[0]

        pmax = jnp.maximum(jnp.maximum(ptrs[0], ptrs[1]),
                           jnp.maximum(ptrs[2], ptrs[3]))

        def flush(c):
            top = drain(c[0], c[1:])
            return ((jnp.int32(0),) * NCH,) + top

        return lax.cond(pmax >= CANDBUF, flush, lambda c: c,
                        (tuple(ptrs),) + top)

    init = ((jnp.int32(0),) * NCH,
            jnp.full((L,), neg), jnp.zeros((L,), jnp.int32),
            jnp.full((L,), neg), jnp.zeros((L,), jnp.int32),
            t0)
    with jax.named_scope("scan"):
        out = lax.fori_loop(0, NV // SG, scan_group, init)
        hik, hii, lok, loi, _ = drain(out[0], out[1:])

    idx_v[pl.ds(0, L)] = hii
    idx_v[pl.ds(L, L)] = loi
    gathers = [
        pltpu.async_copy(
            wdec_hbm.at[idx_v.at[pl.ds(k * GROWS, GROWS)]],
            bufa_v if k % 2 == 0 else bufb_v,
            semg_a if k % 2 == 0 else semg_b)
        for k in range(1)
    ]

    zero = jnp.zeros((L,), jnp.float32)

    def zbody(i, _):
        base = i * (8 * L)
        for u in range(8):
            row_v[pl.ds(base + u * L, L)] = zero
        return 0

    with jax.named_scope("fbuild"):
        lax.fori_loop(0, NV // 8, zbody, 0)
        plsc.store_scatter(row_v, [hii], jnp.maximum(hik, 0.0))
        plsc.store_scatter(row_v, [loi], jnp.maximum(lok, 0.0))
        cpf = pltpu.async_copy(row_v, f_hbm.at[wid], semf)

    vh = jnp.maximum(hik, 0.0)
    vl = jnp.maximum(lok, 0.0)
    ws_all = [vh[r] for r in range(L)] + [vl[r] for r in range(L)]
    cpb.wait()

    nchunks = (2 * L) // GROWS
    scope = jax.named_scope("decode")
    scope.__enter__()
    for k in range(nchunks):
        if k + 1 < nchunks:
            gathers.append(pltpu.async_copy(
                wdec_hbm.at[idx_v.at[pl.ds((k + 1) * GROWS, GROWS)]],
                bufb_v if k % 2 == 0 else bufa_v,
                semg_b if k % 2 == 0 else semg_a))
        gathers[k].wait()
        buf = bufa_v if k % 2 == 0 else bufb_v
        ws = ws_all[k * GROWS:(k + 1) * GROWS]
        src = bdec_v if k == 0 else acc_v

        def jbody(j, _, buf=buf, ws=ws, src=src):
            for q in range(4):
                o = j * (4 * L) + q * L
                t = [ws[r] * buf[r, pl.ds(o, L)] for r in range(GROWS)]
                t = [t[2 * i] + t[2 * i + 1] for i in range(GROWS // 2)]
                t = [t[2 * i] + t[2 * i + 1] for i in range(GROWS // 4)]
                acc_v[pl.ds(o, L)] = src[pl.ds(o, L)] + t[0] + t[1]
            return 0

        lax.fori_loop(0, DM // (4 * L), jbody, 0)

    cpf.wait()
    pltpu.sync_copy(acc_v, recon_hbm.at[wid])
    scope.__exit__(None, None, None)


def _decode_topk(pre, W_dec, b_dec):
    mesh = plsc.VectorSubcoreMesh(
        core_axis_name="c", subcore_axis_name="s",
        num_cores=NC, num_subcores=NS)
    fn = functools.partial(
        pl.kernel,
        out_type=(jax.ShapeDtypeStruct((BT, NF), jnp.float32),
                  jax.ShapeDtypeStruct((BT, DM), jnp.float32)),
        mesh=mesh,
        scratch_types=[
            pltpu.VMEM((NF,), jnp.float32),
            pltpu.VMEM((GROWS, DM), jnp.float32),
            pltpu.VMEM((GROWS, DM), jnp.float32),
            pltpu.VMEM((DM,), jnp.float32),
            pltpu.VMEM((DM,), jnp.float32),
            pltpu.VMEM((2 * L,), jnp.int32),
            pltpu.VMEM((CANDBUF + 6 * L,), jnp.float32),
            pltpu.VMEM((CANDBUF + 6 * L,), jnp.float32),
            pltpu.VMEM((CANDBUF + 6 * L,), jnp.float32),
            pltpu.VMEM((CANDBUF + 6 * L,), jnp.float32),
            pltpu.VMEM((CANDBUF + 6 * L,), jnp.int32),
            pltpu.VMEM((CANDBUF + 6 * L,), jnp.int32),
            pltpu.VMEM((CANDBUF + 6 * L,), jnp.int32),
            pltpu.VMEM((CANDBUF + 6 * L,), jnp.int32),
            pltpu.SemaphoreType.DMA,
            pltpu.SemaphoreType.DMA,
            pltpu.SemaphoreType.DMA,
            pltpu.SemaphoreType.DMA,
        ],
        compiler_params=pltpu.CompilerParams(needs_layout_passes=False),
    )(_sc_body)
    return fn(pre, W_dec, b_dec)


def kernel(x, W_enc, b_enc, W_dec, b_dec):
    pre = _encode(x, W_enc, b_enc, b_dec)
    f, recon = _decode_topk(pre, W_dec, b_dec)
    return (recon, f)


# speedup vs baseline: 1.1110x; 1.0057x over previous
import functools

import jax
import jax.numpy as jnp
from jax import lax
from jax.experimental import pallas as pl
from jax.experimental.pallas import tpu as pltpu
from jax.experimental.pallas import tpu_sc as plsc

DM = 4096
NF = 32768
BT = 32
KTOP = 32
L = 16
NC, NS = 2, 16
NV = NF // L
GRP = 8
GROWS = 8
CANDBUF = 1024

BN = 1024


def _enc_body(x_ref, bdec_ref, w_ref, benc_ref, o_ref):
    xm = x_ref[...] - bdec_ref[...]
    o_ref[...] = (
        jnp.dot(xm, w_ref[...], preferred_element_type=jnp.float32)
        + benc_ref[...]
    )


def _encode(x, W_enc, b_enc, b_dec):
    return pl.pallas_call(
        _enc_body,
        grid=(NF // BN,),
        in_specs=[
            pl.BlockSpec((BT, DM), lambda i: (0, 0)),
            pl.BlockSpec((1, DM), lambda i: (0, 0)),
            pl.BlockSpec((DM, BN), lambda i: (0, i)),
            pl.BlockSpec((1, BN), lambda i: (0, i)),
        ],
        out_specs=pl.BlockSpec((BT, BN), lambda i: (0, i)),
        out_shape=jax.ShapeDtypeStruct((BT, NF), jnp.float32),
    )(x, b_dec.reshape(1, DM), W_enc, b_enc.reshape(1, NF))


def _merge16(hik, hii, lok, loi, sk, si):
    rk = lax.rev(sk, (0,))
    ri = lax.rev(si, (0,))
    p = lok >= rk
    ak = jnp.where(p, lok, rk)
    ai = jnp.where(p, loi, ri)
    ak, ai = plsc.sort_key_val(ak, ai, descending=True)
    rk = lax.rev(ak, (0,))
    ri = lax.rev(ai, (0,))
    p = hik >= rk
    nk = jnp.where(p, hik, rk)
    ni = jnp.where(p, hii, ri)
    mk = jnp.where(p, rk, hik)
    mi = jnp.where(p, ri, hii)
    nk, ni = plsc.sort_key_val(nk, ni, descending=True)
    mk, mi = plsc.sort_key_val(mk, mi, descending=True)
    return nk, ni, mk, mi


def _sc_body(pre_hbm, wdec_hbm, bdec_hbm, f_hbm, recon_hbm,
             row_v, bufa_v, bufb_v, acc_v, bdec_v, idx_v,
             cand0_v, cand1_v, cand2_v, cand3_v,
             candi0_v, candi1_v, candi2_v, candi3_v,
             semg_a, semg_b, semb, semf):
    wid = lax.axis_index("s") * NC + lax.axis_index("c")
    cpb = pltpu.async_copy(bdec_hbm, bdec_v, semb)
    with jax.named_scope("rowdma"):
        pltpu.sync_copy(pre_hbm.at[wid], row_v)

    neg = jnp.float32(-3.0e38)
    lane = lax.iota(jnp.int32, L)

    def boot(i, c):
        ca, cb = c
        base = i * (8 * L)
        for u in range(0, 8, 2):
            ca = jnp.maximum(ca, row_v[pl.ds(base + u * L, L)])
            cb = jnp.maximum(cb, row_v[pl.ds(base + (u + 1) * L, L)])
        return ca, cb

    with jax.named_scope("boot"):
        ca, cb = lax.fori_loop(0, NV // 8, boot,
                               (jnp.full((L,), neg), jnp.full((L,), neg)))
        t0 = lax.reduce_min(jnp.minimum(ca, cb), (0,))

    t0v = jnp.full((L,), t0)
    SG = 16
    NCH = 4
    cands = [cand0_v, cand1_v, cand2_v, cand3_v]
    candis = [candi0_v, candi1_v, candi2_v, candi3_v]

    def drain(ptrs, top):
        ones = lane >= 0
        for c in range(NCH):
            plsc.store_compressed(cands[c].at[pl.ds(ptrs[c], L)],
                                  jnp.full((L,), neg), mask=ones)
        del ones

        def dbody(b, c, cv=None, civ=None):
            hik, hii, lok, loi, thr = c
            v = cv[pl.ds(b * L, L)]
            iv = civ[pl.ds(b * L, L)]

            def do(c):
                hik, hii, lok, loi, _ = c
                sk, si = plsc.sort_key_val(v, iv, descending=True)
                hik, hii, lok, loi = _merge16(hik, hii, lok, loi, sk, si)
                thr = jnp.maximum(t0, lax.reduce_min(lok, (0,)))
                return hik, hii, lok, loi, thr

            vmax = lax.reduce_max(v, (0,))
            return lax.cond(vmax >= c[4], do, lambda c: c, c)

        for c in range(NCH):
            nb = (ptrs[c] + L - 1) // L
            top = lax.fori_loop(
                0, nb,
                functools.partial(dbody, cv=cands[c], civ=candis[c]), top)
        return top

    def scan_group(g, carry):
        ptrs, top = list(carry[0]), carry[1:]
        base = g * (SG * L)
        for u in range(SG):
            c = u % NCH
            off = base + u * L
            v = row_v[pl.ds(off, L)]
            m = v >= t0v
            plsc.store_compressed(cands[c].at[pl.ds(ptrs[c], L)], v, mask=m)
            plsc.store_compressed(candis[c].at[pl.ds(ptrs[c], L)],
                                  lane + off, mask=m)
            ptrs[c] = ptrs[c] + plsc.all_reduce_population_count(m)[0]

        pmax = jnp.maximum(jnp.maximum(ptrs[0], ptrs[1]),
                           jnp.maximum(ptrs[2], ptrs[3]))

        def flush(c):
            top = drain(c[0], c[1:])
            return ((jnp.int32(0),) * NCH,) + top

        return lax.cond(pmax >= CANDBUF, flush, lambda c: c,
                        (tuple(ptrs),) + top)

    init = ((jnp.int32(0),) * NCH,
            jnp.full((L,), neg), jnp.zeros((L,), jnp.int32),
            jnp.full((L,), neg), jnp.zeros((L,), jnp.int32),
            t0)
    with jax.named_scope("scan"):
        out = lax.fori_loop(0, NV // SG, scan_group, init)
        hik, hii, lok, loi, _ = drain(out[0], out[1:])

    idx_v[pl.ds(0, L)] = hii
    idx_v[pl.ds(L, L)] = loi
    gathers = [
        pltpu.async_copy(
            wdec_hbm.at[idx_v.at[pl.ds(k * GROWS, GROWS)]],
            bufa_v if k % 2 == 0 else bufb_v,
            semg_a if k % 2 == 0 else semg_b)
        for k in range(1)
    ]

    zero = jnp.zeros((L,), jnp.float32)

    def zbody(i, _):
        base = i * (8 * L)
        for u in range(8):
            row_v[pl.ds(base + u * L, L)] = zero
        return 0

    with jax.named_scope("fbuild"):
        lax.fori_loop(0, NV // 8, zbody, 0)
        plsc.store_scatter(row_v, [hii], jnp.maximum(hik, 0.0))
        plsc.store_scatter(row_v, [loi], jnp.maximum(lok, 0.0))
        cpf = pltpu.async_copy(row_v, f_hbm.at[wid], semf)

    vh = jnp.maximum(hik, 0.0)
    vl = jnp.maximum(lok, 0.0)
    ws_all = [vh[r] for r in range(L)] + [vl[r] for r in range(L)]
    cpb.wait()

    nchunks = (2 * L) // GROWS
    scope = jax.named_scope("decode")
    scope.__enter__()
    for k in range(nchunks):
        if k + 1 < nchunks:
            gathers.append(pltpu.async_copy(
                wdec_hbm.at[idx_v.at[pl.ds((k + 1) * GROWS, GROWS)]],
                bufb_v if k % 2 == 0 else bufa_v,
                semg_b if k % 2 == 0 else semg_a))
        gathers[k].wait()
        buf = bufa_v if k % 2 == 0 else bufb_v
        ws = ws_all[k * GROWS:(k + 1) * GROWS]
        src = bdec_v if k == 0 else acc_v

        def jbody(j, _, buf=buf, ws=ws, src=src):
            for q in range(4):
                o = j * (4 * L) + q * L
                t = [ws[r] * buf[r, pl.ds(o, L)] for r in range(GROWS)]
                t = [t[2 * i] + t[2 * i + 1] for i in range(GROWS // 2)]
                t = [t[2 * i] + t[2 * i + 1] for i in range(GROWS // 4)]
                acc_v[pl.ds(o, L)] = src[pl.ds(o, L)] + t[0] + t[1]
            return 0

        lax.fori_loop(0, DM // (4 * L), jbody, 0)

    cpf.wait()
    pltpu.sync_copy(acc_v, recon_hbm.at[wid])
    scope.__exit__(None, None, None)


def _decode_topk(pre, W_dec, b_dec):
    mesh = plsc.VectorSubcoreMesh(
        core_axis_name="c", subcore_axis_name="s",
        num_cores=NC, num_subcores=NS)
    fn = functools.partial(
        pl.kernel,
        out_type=(jax.ShapeDtypeStruct((BT, NF), jnp.float32),
                  jax.ShapeDtypeStruct((BT, DM), jnp.float32)),
        mesh=mesh,
        scratch_types=[
            pltpu.VMEM((NF,), jnp.float32),
            pltpu.VMEM((GROWS, DM), jnp.float32),
            pltpu.VMEM((GROWS, DM), jnp.float32),
            pltpu.VMEM((DM,), jnp.float32),
            pltpu.VMEM((DM,), jnp.float32),
            pltpu.VMEM((2 * L,), jnp.int32),
            pltpu.VMEM((CANDBUF + 6 * L,), jnp.float32),
            pltpu.VMEM((CANDBUF + 6 * L,), jnp.float32),
            pltpu.VMEM((CANDBUF + 6 * L,), jnp.float32),
            pltpu.VMEM((CANDBUF + 6 * L,), jnp.float32),
            pltpu.VMEM((CANDBUF + 6 * L,), jnp.int32),
            pltpu.VMEM((CANDBUF + 6 * L,), jnp.int32),
            pltpu.VMEM((CANDBUF + 6 * L,), jnp.int32),
            pltpu.VMEM((CANDBUF + 6 * L,), jnp.int32),
            pltpu.SemaphoreType.DMA,
            pltpu.SemaphoreType.DMA,
            pltpu.SemaphoreType.DMA,
            pltpu.SemaphoreType.DMA,
        ],
        compiler_params=pltpu.CompilerParams(needs_layout_passes=False),
    )(_sc_body)
    return fn(pre, W_dec, b_dec)


def kernel(x, W_enc, b_enc, W_dec, b_dec):
    pre = _encode(x, W_enc, b_enc, b_dec)
    f, recon = _decode_topk(pre, W_dec, b_dec)
    return (recon, f)


# speedup vs baseline: 1.1139x; 1.0027x over previous
import functools

import jax
import jax.numpy as jnp
from jax import lax
from jax.experimental import pallas as pl
from jax.experimental.pallas import tpu as pltpu
from jax.experimental.pallas import tpu_sc as plsc

DM = 4096
NF = 32768
NF2 = NF // 2
BT = 32
L = 16
NC, NS = 2, 16
NV2 = NF2 // L
GROWS = 8
CANDBUF = 512
NCH = 4
SG = 16

BN = 1024


def _enc_body(x_ref, bdec_ref, w_ref, benc_ref, o_ref):
    xm = x_ref[...] - bdec_ref[...]
    o_ref[...] = (
        jnp.dot(xm, w_ref[...], preferred_element_type=jnp.float32)
        + benc_ref[...]
    )


def _encode_half(x, W_enc, b_enc, b_dec, h):
    nblk = NF2 // BN
    return pl.pallas_call(
        _enc_body,
        grid=(nblk,),
        in_specs=[
            pl.BlockSpec((BT, DM), lambda i: (0, 0)),
            pl.BlockSpec((1, DM), lambda i: (0, 0)),
            pl.BlockSpec((DM, BN), lambda i, h=h: (0, h * nblk + i)),
            pl.BlockSpec((1, BN), lambda i, h=h: (0, h * nblk + i)),
        ],
        out_specs=pl.BlockSpec((BT, BN), lambda i: (0, i)),
        out_shape=jax.ShapeDtypeStruct((BT, NF2), jnp.float32),
        name=f"enc{h}",
    )(x, b_dec.reshape(1, DM), W_enc, b_enc.reshape(1, NF))


def _merge16(hik, hii, lok, loi, sk, si):
    rk = lax.rev(sk, (0,))
    ri = lax.rev(si, (0,))
    p = lok >= rk
    ak = jnp.where(p, lok, rk)
    ai = jnp.where(p, loi, ri)
    ak, ai = plsc.sort_key_val(ak, ai, descending=True)
    rk = lax.rev(ak, (0,))
    ri = lax.rev(ai, (0,))
    p = hik >= rk
    nk = jnp.where(p, hik, rk)
    ni = jnp.where(p, hii, ri)
    mk = jnp.where(p, rk, hik)
    mi = jnp.where(p, ri, hii)
    nk, ni = plsc.sort_key_val(nk, ni, descending=True)
    mk, mi = plsc.sort_key_val(mk, mi, descending=True)
    return nk, ni, mk, mi


_NEG = -3.0e38


def _compact_topk(row_v, cands, candis, t0, idx_base, init_top):
    lane = lax.iota(jnp.int32, L)
    neg = jnp.float32(_NEG)
    t0v = jnp.full((L,), t0)

    def drain(ptrs, top):
        ones = lane >= 0
        for c in range(NCH):
            plsc.store_compressed(cands[c].at[pl.ds(ptrs[c], L)],
                                  jnp.full((L,), neg), mask=ones)

        def dbody(b, c, cv=None, civ=None):
            v = cv[pl.ds(b * L, L)]
            iv = civ[pl.ds(b * L, L)]

            def do(c):
                hik, hii, lok, loi, _ = c
                sk, si = plsc.sort_key_val(v, iv, descending=True)
                hik, hii, lok, loi = _merge16(hik, hii, lok, loi, sk, si)
                thr = jnp.maximum(t0, lax.reduce_min(lok, (0,)))
                return hik, hii, lok, loi, thr

            vmax = lax.reduce_max(v, (0,))
            return lax.cond(vmax >= c[4], do, lambda c: c, c)

        for c in range(NCH):
            nb = (ptrs[c] + L - 1) // L
            top = lax.fori_loop(
                0, nb,
                functools.partial(dbody, cv=cands[c], civ=candis[c]), top)
        return top

    def scan_group(g, carry):
        ptrs, top = list(carry[0]), carry[1:]
        base = g * (SG * L)
        for u in range(SG):
            c = u % NCH
            off = base + u * L
            v = row_v[pl.ds(off, L)]
            m = v >= t0v
            plsc.store_compressed(cands[c].at[pl.ds(ptrs[c], L)], v, mask=m)
            plsc.store_compressed(candis[c].at[pl.ds(ptrs[c], L)],
                                  lane + (off + idx_base), mask=m)
            ptrs[c] = ptrs[c] + plsc.all_reduce_population_count(m)[0]

        pmax = jnp.maximum(jnp.maximum(ptrs[0], ptrs[1]),
                           jnp.maximum(ptrs[2], ptrs[3]))

        def flush(c):
            top = drain(c[0], c[1:])
            return ((jnp.int32(0),) * NCH,) + top

        return lax.cond(pmax >= CANDBUF, flush, lambda c: c,
                        (tuple(ptrs),) + top)

    hik, hii, lok, loi = init_top
    thr0 = jnp.maximum(t0, lax.reduce_min(lok, (0,)))
    init = ((jnp.int32(0),) * NCH, hik, hii, lok, loi, thr0)
    out = lax.fori_loop(0, NV2 // SG, scan_group, init)
    hik, hii, lok, loi, _ = drain(out[0], out[1:])
    return hik, hii, lok, loi


_CAND_SCRATCH = (
    [pltpu.VMEM((CANDBUF + 6 * L,), jnp.float32)] * NCH
    + [pltpu.VMEM((CANDBUF + 6 * L,), jnp.int32)] * NCH
)


def _sc_scan_body(pre_hbm, vout_hbm, iout_hbm,
                  row_v, stv_v, sti_v,
                  cand0_v, cand1_v, cand2_v, cand3_v,
                  candi0_v, candi1_v, candi2_v, candi3_v):
    wid = lax.axis_index("s") * NC + lax.axis_index("c")
    pltpu.sync_copy(pre_hbm.at[wid], row_v)

    neg = jnp.float32(_NEG)

    def boot(i, c):
        ca, cb = c
        base = i * (8 * L)
        for u in range(0, 8, 2):
            ca = jnp.maximum(ca, row_v[pl.ds(base + u * L, L)])
            cb = jnp.maximum(cb, row_v[pl.ds(base + (u + 1) * L, L)])
        return ca, cb

    ca, cb = lax.fori_loop(0, NV2 // 8, boot,
                           (jnp.full((L,), neg), jnp.full((L,), neg)))
    t0 = lax.reduce_min(jnp.minimum(ca, cb), (0,))

    init_top = (jnp.full((L,), neg), jnp.zeros((L,), jnp.int32),
                jnp.full((L,), neg), jnp.zeros((L,), jnp.int32))
    hik, hii, lok, loi = _compact_topk(
        row_v, [cand0_v, cand1_v, cand2_v, cand3_v],
        [candi0_v, candi1_v, candi2_v, candi3_v], t0, 0, init_top)

    stv_v[pl.ds(0, L)] = hik
    stv_v[pl.ds(L, L)] = lok
    sti_v[pl.ds(0, L)] = hii
    sti_v[pl.ds(L, L)] = loi
    pltpu.sync_copy(stv_v, vout_hbm.at[wid])
    pltpu.sync_copy(sti_v, iout_hbm.at[wid])


def _sc_scan(pre1):
    mesh = plsc.VectorSubcoreMesh(
        core_axis_name="c", subcore_axis_name="s",
        num_cores=NC, num_subcores=NS)
    fn = functools.partial(
        pl.kernel,
        out_type=(jax.ShapeDtypeStruct((BT, 2 * L), jnp.float32),
                  jax.ShapeDtypeStruct((BT, 2 * L), jnp.int32)),
        mesh=mesh,
        scratch_types=[
            pltpu.VMEM((NF2,), jnp.float32),
            pltpu.VMEM((2 * L,), jnp.float32),
            pltpu.VMEM((2 * L,), jnp.int32),
        ] + _CAND_SCRATCH,
        compiler_params=pltpu.CompilerParams(needs_layout_passes=False),
        name="sc_scan",
    )(_sc_scan_body)
    return fn(pre1)


def _sc_final_body(pre_hbm, v1_hbm, i1_hbm, wdec_hbm, bdec_hbm,
                   f_hbm, recon_hbm,
                   row_v, fst_v, bufa_v, bufb_v, acc_v, bdec_v, idx_v,
                   stv_v, sti_v,
                   cand0_v, cand1_v, cand2_v, cand3_v,
                   candi0_v, candi1_v, candi2_v, candi3_v,
                   semg_a, semg_b, semb, semf, semr):
    wid = lax.axis_index("s") * NC + lax.axis_index("c")
    cpb = pltpu.async_copy(bdec_hbm, bdec_v, semb)
    cpr = pltpu.async_copy(pre_hbm.at[wid], row_v, semr)
    pltpu.sync_copy(v1_hbm.at[wid], stv_v)
    pltpu.sync_copy(i1_hbm.at[wid], sti_v)

    zero = jnp.zeros((L,), jnp.float32)

    def zbody(i, _):
        base = i * (8 * L)
        for u in range(8):
            fst_v[pl.ds(base + u * L, L)] = zero
        return 0

    lax.fori_loop(0, NF // (8 * L), zbody, 0)

    hik = stv_v[pl.ds(0, L)]
    lok = stv_v[pl.ds(L, L)]
    hii = sti_v[pl.ds(0, L)]
    loi = sti_v[pl.ds(L, L)]
    t0 = lax.reduce_min(lok, (0,))
    cpr.wait()

    hik, hii, lok, loi = _compact_topk(
        row_v, [cand0_v, cand1_v, cand2_v, cand3_v],
        [candi0_v, candi1_v, candi2_v, candi3_v], t0, NF2,
        (hik, hii, lok, loi))

    idx_v[pl.ds(0, L)] = hii
    idx_v[pl.ds(L, L)] = loi
    gathers = [
        pltpu.async_copy(
            wdec_hbm.at[idx_v.at[pl.ds(0, GROWS)]], bufa_v, semg_a)
    ]

    plsc.store_scatter(fst_v, [hii], jnp.maximum(hik, 0.0))
    plsc.store_scatter(fst_v, [loi], jnp.maximum(lok, 0.0))
    cpf = pltpu.async_copy(fst_v, f_hbm.at[wid], semf)

    vh = jnp.maximum(hik, 0.0)
    vl = jnp.maximum(lok, 0.0)
    ws_all = [vh[r] for r in range(L)] + [vl[r] for r in range(L)]
    cpb.wait()

    nchunks = (2 * L) // GROWS
    for k in range(nchunks):
        if k + 1 < nchunks:
            gathers.append(pltpu.async_copy(
                wdec_hbm.at[idx_v.at[pl.ds((k + 1) * GROWS, GROWS)]],
                bufb_v if k % 2 == 0 else bufa_v,
                semg_b if k % 2 == 0 else semg_a))
        gathers[k].wait()
        buf = bufa_v if k % 2 == 0 else bufb_v
        ws = ws_all[k * GROWS:(k + 1) * GROWS]
        src = bdec_v if k == 0 else acc_v

        def jbody(j, _, buf=buf, ws=ws, src=src):
            for q in range(4):
                o = j * (4 * L) + q * L
                t = [ws[r] * buf[r, pl.ds(o, L)] for r in range(GROWS)]
                t = [t[2 * i] + t[2 * i + 1] for i in range(GROWS // 2)]
                t = [t[2 * i] + t[2 * i + 1] for i in range(GROWS // 4)]
                acc_v[pl.ds(o, L)] = src[pl.ds(o, L)] + t[0] + t[1]
            return 0

        lax.fori_loop(0, DM // (4 * L), jbody, 0)

    cpf.wait()
    pltpu.sync_copy(acc_v, recon_hbm.at[wid])


def _sc_final(pre2, v1, i1, W_dec, b_dec):
    mesh = plsc.VectorSubcoreMesh(
        core_axis_name="c", subcore_axis_name="s",
        num_cores=NC, num_subcores=NS)
    fn = functools.partial(
        pl.kernel,
        out_type=(jax.ShapeDtypeStruct((BT, NF), jnp.float32),
                  jax.ShapeDtypeStruct((BT, DM), jnp.float32)),
        mesh=mesh,
        scratch_types=[
            pltpu.VMEM((NF2,), jnp.float32),
            pltpu.VMEM((NF,), jnp.float32),
            pltpu.VMEM((GROWS, DM), jnp.float32),
            pltpu.VMEM((GROWS, DM), jnp.float32),
            pltpu.VMEM((DM,), jnp.float32),
            pltpu.VMEM((DM,), jnp.float32),
            pltpu.VMEM((2 * L,), jnp.int32),
            pltpu.VMEM((2 * L,), jnp.float32),
            pltpu.VMEM((2 * L,), jnp.int32),
        ] + _CAND_SCRATCH + [
            pltpu.SemaphoreType.DMA,
            pltpu.SemaphoreType.DMA,
            pltpu.SemaphoreType.DMA,
            pltpu.SemaphoreType.DMA,
            pltpu.SemaphoreType.DMA,
        ],
        compiler_params=pltpu.CompilerParams(needs_layout_passes=False),
        name="sc_final",
    )(_sc_final_body)
    return fn(pre2, v1, i1, W_dec, b_dec)


def kernel(x, W_enc, b_enc, W_dec, b_dec):
    pre1 = _encode_half(x, W_enc, b_enc, b_dec, 0)
    v1, i1 = _sc_scan(pre1)
    pre2 = _encode_half(x, W_enc, b_enc, b_dec, 1)
    f, recon = _sc_final(pre2, v1, i1, W_dec, b_dec)
    return (recon, f)


# speedup vs baseline: 1.1239x; 1.0090x over previous
import functools

import jax
import jax.numpy as jnp
from jax import lax
from jax.experimental import pallas as pl
from jax.experimental.pallas import tpu as pltpu
from jax.experimental.pallas import tpu_sc as plsc

DM = 4096
NF = 32768
NF2 = NF // 2
BT = 32
L = 16
NC, NS = 2, 16
NV2 = NF2 // L
GROWS = 8
CANDBUF = 512
NCH = 4
SG = 16

BN = 512


def _enc_body(x_ref, bdec_ref, w_ref, benc_ref, o_ref):
    xm = x_ref[...] - bdec_ref[...]
    o_ref[...] = (
        jnp.dot(xm, w_ref[...], preferred_element_type=jnp.float32)
        + benc_ref[...]
    )


def _encode_half(x, W_enc, b_enc, b_dec, h):
    nblk = NF2 // BN
    return pl.pallas_call(
        _enc_body,
        grid=(nblk,),
        in_specs=[
            pl.BlockSpec((BT, DM), lambda i: (0, 0)),
            pl.BlockSpec((1, DM), lambda i: (0, 0)),
            pl.BlockSpec((DM, BN), lambda i, h=h: (0, h * nblk + i)),
            pl.BlockSpec((1, BN), lambda i, h=h: (0, h * nblk + i)),
        ],
        out_specs=pl.BlockSpec((BT, BN), lambda i: (0, i)),
        out_shape=jax.ShapeDtypeStruct((BT, NF2), jnp.float32),
        name=f"enc{h}",
    )(x, b_dec.reshape(1, DM), W_enc, b_enc.reshape(1, NF))


def _merge16(hik, hii, lok, loi, sk, si):
    rk = lax.rev(sk, (0,))
    ri = lax.rev(si, (0,))
    p = lok >= rk
    ak = jnp.where(p, lok, rk)
    ai = jnp.where(p, loi, ri)
    ak, ai = plsc.sort_key_val(ak, ai, descending=True)
    rk = lax.rev(ak, (0,))
    ri = lax.rev(ai, (0,))
    p = hik >= rk
    nk = jnp.where(p, hik, rk)
    ni = jnp.where(p, hii, ri)
    mk = jnp.where(p, rk, hik)
    mi = jnp.where(p, ri, hii)
    nk, ni = plsc.sort_key_val(nk, ni, descending=True)
    mk, mi = plsc.sort_key_val(mk, mi, descending=True)
    return nk, ni, mk, mi


_NEG = -3.0e38


def _compact_topk(row_v, cands, candis, t0, idx_base, init_top):
    lane = lax.iota(jnp.int32, L)
    neg = jnp.float32(_NEG)
    t0v = jnp.full((L,), t0)

    def drain(ptrs, top):
        ones = lane >= 0
        for c in range(NCH):
            plsc.store_compressed(cands[c].at[pl.ds(ptrs[c], L)],
                                  jnp.full((L,), neg), mask=ones)

        def dbody(b, c, cv=None, civ=None):
            v = cv[pl.ds(b * L, L)]
            iv = civ[pl.ds(b * L, L)]

            def do(c):
                hik, hii, lok, loi, _ = c
                sk, si = plsc.sort_key_val(v, iv, descending=True)
                hik, hii, lok, loi = _merge16(hik, hii, lok, loi, sk, si)
                thr = jnp.maximum(t0, lax.reduce_min(lok, (0,)))
                return hik, hii, lok, loi, thr

            vmax = lax.reduce_max(v, (0,))
            return lax.cond(vmax >= c[4], do, lambda c: c, c)

        for c in range(NCH):
            nb = (ptrs[c] + L - 1) // L
            top = lax.fori_loop(
                0, nb,
                functools.partial(dbody, cv=cands[c], civ=candis[c]), top)
        return top

    def scan_group(g, carry):
        ptrs, top = list(carry[0]), carry[1:]
        base = g * (SG * L)
        for u in range(SG):
            c = u % NCH
            off = base + u * L
            v = row_v[pl.ds(off, L)]
            m = v >= t0v
            plsc.store_compressed(cands[c].at[pl.ds(ptrs[c], L)], v, mask=m)
            plsc.store_compressed(candis[c].at[pl.ds(ptrs[c], L)],
                                  lane + (off + idx_base), mask=m)
            ptrs[c] = ptrs[c] + plsc.all_reduce_population_count(m)[0]

        pmax = jnp.maximum(jnp.maximum(ptrs[0], ptrs[1]),
                           jnp.maximum(ptrs[2], ptrs[3]))

        def flush(c):
            top = drain(c[0], c[1:])
            return ((jnp.int32(0),) * NCH,) + top

        return lax.cond(pmax >= CANDBUF, flush, lambda c: c,
                        (tuple(ptrs),) + top)

    hik, hii, lok, loi = init_top
    thr0 = jnp.maximum(t0, lax.reduce_min(lok, (0,)))
    init = ((jnp.int32(0),) * NCH, hik, hii, lok, loi, thr0)
    out = lax.fori_loop(0, NV2 // SG, scan_group, init)
    hik, hii, lok, loi, _ = drain(out[0], out[1:])
    return hik, hii, lok, loi


_CAND_SCRATCH = (
    [pltpu.VMEM((CANDBUF + 6 * L,), jnp.float32)] * NCH
    + [pltpu.VMEM((CANDBUF + 6 * L,), jnp.int32)] * NCH
)


def _sc_scan_body(pre_hbm, vout_hbm, iout_hbm,
                  row_v, stv_v, sti_v,
                  cand0_v, cand1_v, cand2_v, cand3_v,
                  candi0_v, candi1_v, candi2_v, candi3_v):
    wid = lax.axis_index("s") * NC + lax.axis_index("c")
    pltpu.sync_copy(pre_hbm.at[wid], row_v)

    neg = jnp.float32(_NEG)

    def boot(i, c):
        ca, cb = c
        base = i * (8 * L)
        for u in range(0, 8, 2):
            ca = jnp.maximum(ca, row_v[pl.ds(base + u * L, L)])
            cb = jnp.maximum(cb, row_v[pl.ds(base + (u + 1) * L, L)])
        return ca, cb

    ca, cb = lax.fori_loop(0, NV2 // 8, boot,
                           (jnp.full((L,), neg), jnp.full((L,), neg)))
    t0 = lax.reduce_min(jnp.minimum(ca, cb), (0,))

    init_top = (jnp.full((L,), neg), jnp.zeros((L,), jnp.int32),
                jnp.full((L,), neg), jnp.zeros((L,), jnp.int32))
    hik, hii, lok, loi = _compact_topk(
        row_v, [cand0_v, cand1_v, cand2_v, cand3_v],
        [candi0_v, candi1_v, candi2_v, candi3_v], t0, 0, init_top)

    stv_v[pl.ds(0, L)] = hik
    stv_v[pl.ds(L, L)] = lok
    sti_v[pl.ds(0, L)] = hii
    sti_v[pl.ds(L, L)] = loi
    pltpu.sync_copy(stv_v, vout_hbm.at[wid])
    pltpu.sync_copy(sti_v, iout_hbm.at[wid])


def _sc_scan(pre1):
    mesh = plsc.VectorSubcoreMesh(
        core_axis_name="c", subcore_axis_name="s",
        num_cores=NC, num_subcores=NS)
    fn = functools.partial(
        pl.kernel,
        out_type=(jax.ShapeDtypeStruct((BT, 2 * L), jnp.float32),
                  jax.ShapeDtypeStruct((BT, 2 * L), jnp.int32)),
        mesh=mesh,
        scratch_types=[
            pltpu.VMEM((NF2,), jnp.float32),
            pltpu.VMEM((2 * L,), jnp.float32),
            pltpu.VMEM((2 * L,), jnp.int32),
        ] + _CAND_SCRATCH,
        compiler_params=pltpu.CompilerParams(needs_layout_passes=False),
        name="sc_scan",
    )(_sc_scan_body)
    return fn(pre1)


def _sc_final_body(pre_hbm, v1_hbm, i1_hbm, wdec_hbm, bdec_hbm,
                   f_hbm, recon_hbm,
                   row_v, fst_v, bufa_v, bufb_v, acc_v, bdec_v, idx_v,
                   stv_v, sti_v,
                   cand0_v, cand1_v, cand2_v, cand3_v,
                   candi0_v, candi1_v, candi2_v, candi3_v,
                   semg_a, semg_b, semb, semf, semr):
    wid = lax.axis_index("s") * NC + lax.axis_index("c")
    cpb = pltpu.async_copy(bdec_hbm, bdec_v, semb)
    cpr = pltpu.async_copy(pre_hbm.at[wid], row_v, semr)
    pltpu.sync_copy(v1_hbm.at[wid], stv_v)
    pltpu.sync_copy(i1_hbm.at[wid], sti_v)

    zero = jnp.zeros((L,), jnp.float32)

    def zbody(i, _):
        base = i * (8 * L)
        for u in range(8):
            fst_v[pl.ds(base + u * L, L)] = zero
        return 0

    lax.fori_loop(0, NF // (8 * L), zbody, 0)

    hik = stv_v[pl.ds(0, L)]
    lok = stv_v[pl.ds(L, L)]
    hii = sti_v[pl.ds(0, L)]
    loi = sti_v[pl.ds(L, L)]
    t0 = lax.reduce_min(lok, (0,))
    cpr.wait()

    hik, hii, lok, loi = _compact_topk(
        row_v, [cand0_v, cand1_v, cand2_v, cand3_v],
        [candi0_v, candi1_v, candi2_v, candi3_v], t0, NF2,
        (hik, hii, lok, loi))

    idx_v[pl.ds(0, L)] = hii
    idx_v[pl.ds(L, L)] = loi
    gathers = [
        pltpu.async_copy(
            wdec_hbm.at[idx_v.at[pl.ds(0, GROWS)]], bufa_v, semg_a)
    ]

    plsc.store_scatter(fst_v, [hii], jnp.maximum(hik, 0.0))
    plsc.store_scatter(fst_v, [loi], jnp.maximum(lok, 0.0))
    cpf = pltpu.async_copy(fst_v, f_hbm.at[wid], semf)

    vh = jnp.maximum(hik, 0.0)
    vl = jnp.maximum(lok, 0.0)
    ws_all = [vh[r] for r in range(L)] + [vl[r] for r in range(L)]
    cpb.wait()

    nchunks = (2 * L) // GROWS
    for k in range(nchunks):
        if k + 1 < nchunks:
            gathers.append(pltpu.async_copy(
                wdec_hbm.at[idx_v.at[pl.ds((k + 1) * GROWS, GROWS)]],
                bufb_v if k % 2 == 0 else bufa_v,
                semg_b if k % 2 == 0 else semg_a))
        gathers[k].wait()
        buf = bufa_v if k % 2 == 0 else bufb_v
        ws = ws_all[k * GROWS:(k + 1) * GROWS]
        src = bdec_v if k == 0 else acc_v

        def jbody(j, _, buf=buf, ws=ws, src=src):
            for q in range(4):
                o = j * (4 * L) + q * L
                t = [ws[r] * buf[r, pl.ds(o, L)] for r in range(GROWS)]
                t = [t[2 * i] + t[2 * i + 1] for i in range(GROWS // 2)]
                t = [t[2 * i] + t[2 * i + 1] for i in range(GROWS // 4)]
                acc_v[pl.ds(o, L)] = src[pl.ds(o, L)] + t[0] + t[1]
            return 0

        lax.fori_loop(0, DM // (4 * L), jbody, 0)

    cpf.wait()
    pltpu.sync_copy(acc_v, recon_hbm.at[wid])


def _sc_final(pre2, v1, i1, W_dec, b_dec):
    mesh = plsc.VectorSubcoreMesh(
        core_axis_name="c", subcore_axis_name="s",
        num_cores=NC, num_subcores=NS)
    fn = functools.partial(
        pl.kernel,
        out_type=(jax.ShapeDtypeStruct((BT, NF), jnp.float32),
                  jax.ShapeDtypeStruct((BT, DM), jnp.float32)),
        mesh=mesh,
        scratch_types=[
            pltpu.VMEM((NF2,), jnp.float32),
            pltpu.VMEM((NF,), jnp.float32),
            pltpu.VMEM((GROWS, DM), jnp.float32),
            pltpu.VMEM((GROWS, DM), jnp.float32),
            pltpu.VMEM((DM,), jnp.float32),
            pltpu.VMEM((DM,), jnp.float32),
            pltpu.VMEM((2 * L,), jnp.int32),
            pltpu.VMEM((2 * L,), jnp.float32),
            pltpu.VMEM((2 * L,), jnp.int32),
        ] + _CAND_SCRATCH + [
            pltpu.SemaphoreType.DMA,
            pltpu.SemaphoreType.DMA,
            pltpu.SemaphoreType.DMA,
            pltpu.SemaphoreType.DMA,
            pltpu.SemaphoreType.DMA,
        ],
        compiler_params=pltpu.CompilerParams(needs_layout_passes=False),
        name="sc_final",
    )(_sc_final_body)
    return fn(pre2, v1, i1, W_dec, b_dec)


def kernel(x, W_enc, b_enc, W_dec, b_dec):
    pre1 = _encode_half(x, W_enc, b_enc, b_dec, 0)
    v1, i1 = _sc_scan(pre1)
    pre2 = _encode_half(x, W_enc, b_enc, b_dec, 1)
    f, recon = _sc_final(pre2, v1, i1, W_dec, b_dec)
    return (recon, f)


# speedup vs baseline: 1.1481x; 1.0215x over previous
import functools

import jax
import jax.numpy as jnp
from jax import lax
from jax.experimental import pallas as pl
from jax.experimental.pallas import tpu as pltpu
from jax.experimental.pallas import tpu_sc as plsc

DM = 4096
NF = 32768
NFA = 26624
NFB = NF - NFA
BT = 32
L = 16
NC, NS = 2, 16
NVA = NFA // L
NVB = NFB // L
GROWS = 8
CANDBUF = 512
NCH = 4
SG = 16

BN = 512


def _enc_body(x_ref, bdec_ref, w_ref, benc_ref, o_ref):
    xm = x_ref[...] - bdec_ref[...]
    o_ref[...] = (
        jnp.dot(xm, w_ref[...], preferred_element_type=jnp.float32)
        + benc_ref[...]
    )


def _encode_part(x, W_enc, b_enc, b_dec, start, width, tag):
    nblk = width // BN
    blk0 = start // BN
    return pl.pallas_call(
        _enc_body,
        grid=(nblk,),
        in_specs=[
            pl.BlockSpec((BT, DM), lambda i: (0, 0)),
            pl.BlockSpec((1, DM), lambda i: (0, 0)),
            pl.BlockSpec((DM, BN), lambda i, b=blk0: (0, b + i)),
            pl.BlockSpec((1, BN), lambda i, b=blk0: (0, b + i)),
        ],
        out_specs=pl.BlockSpec((BT, BN), lambda i: (0, i)),
        out_shape=jax.ShapeDtypeStruct((BT, width), jnp.float32),
        name=f"enc{tag}",
    )(x, b_dec.reshape(1, DM), W_enc, b_enc.reshape(1, NF))


def _merge16(hik, hii, lok, loi, sk, si):
    rk = lax.rev(sk, (0,))
    ri = lax.rev(si, (0,))
    p = lok >= rk
    ak = jnp.where(p, lok, rk)
    ai = jnp.where(p, loi, ri)
    ak, ai = plsc.sort_key_val(ak, ai, descending=True)
    rk = lax.rev(ak, (0,))
    ri = lax.rev(ai, (0,))
    p = hik >= rk
    nk = jnp.where(p, hik, rk)
    ni = jnp.where(p, hii, ri)
    mk = jnp.where(p, rk, hik)
    mi = jnp.where(p, ri, hii)
    nk, ni = plsc.sort_key_val(nk, ni, descending=True)
    mk, mi = plsc.sort_key_val(mk, mi, descending=True)
    return nk, ni, mk, mi


_NEG = -3.0e38


def _compact_topk(row_v, nv, cands, candis, t0, idx_base, init_top):
    lane = lax.iota(jnp.int32, L)
    neg = jnp.float32(_NEG)
    t0v = jnp.full((L,), t0)

    def drain(ptrs, top):
        ones = lane >= 0
        for c in range(NCH):
            plsc.store_compressed(cands[c].at[pl.ds(ptrs[c], L)],
                                  jnp.full((L,), neg), mask=ones)

        def dbody(b, c, cv=None, civ=None):
            v = cv[pl.ds(b * L, L)]
            iv = civ[pl.ds(b * L, L)]

            def do(c):
                hik, hii, lok, loi, _ = c
                sk, si = plsc.sort_key_val(v, iv, descending=True)
                hik, hii, lok, loi = _merge16(hik, hii, lok, loi, sk, si)
                thr = jnp.maximum(t0, lax.reduce_min(lok, (0,)))
                return hik, hii, lok, loi, thr

            vmax = lax.reduce_max(v, (0,))
            return lax.cond(vmax >= c[4], do, lambda c: c, c)

        for c in range(NCH):
            nb = (ptrs[c] + L - 1) // L
            top = lax.fori_loop(
                0, nb,
                functools.partial(dbody, cv=cands[c], civ=candis[c]), top)
        return top

    def scan_group(g, carry):
        ptrs, top = list(carry[0]), carry[1:]
        base = g * (SG * L)
        for u in range(SG):
            c = u % NCH
            off = base + u * L
            v = row_v[pl.ds(off, L)]
            m = v >= t0v
            plsc.store_compressed(cands[c].at[pl.ds(ptrs[c], L)], v, mask=m)
            plsc.store_compressed(candis[c].at[pl.ds(ptrs[c], L)],
                                  lane + (off + idx_base), mask=m)
            ptrs[c] = ptrs[c] + plsc.all_reduce_population_count(m)[0]

        pmax = jnp.maximum(jnp.maximum(ptrs[0], ptrs[1]),
                           jnp.maximum(ptrs[2], ptrs[3]))

        def flush(c):
            top = drain(c[0], c[1:])
            return ((jnp.int32(0),) * NCH,) + top

        return lax.cond(pmax >= CANDBUF, flush, lambda c: c,
                        (tuple(ptrs),) + top)

    hik, hii, lok, loi = init_top
    thr0 = jnp.maximum(t0, lax.reduce_min(lok, (0,)))
    init = ((jnp.int32(0),) * NCH, hik, hii, lok, loi, thr0)
    out = lax.fori_loop(0, nv // SG, scan_group, init)
    hik, hii, lok, loi, _ = drain(out[0], out[1:])
    return hik, hii, lok, loi


_CAND_SCRATCH = (
    [pltpu.VMEM((CANDBUF + 6 * L,), jnp.float32)] * NCH
    + [pltpu.VMEM((CANDBUF + 6 * L,), jnp.int32)] * NCH
)


def _sc_scan_body(pre_hbm, vout_hbm, iout_hbm,
                  row_v, stv_v, sti_v,
                  cand0_v, cand1_v, cand2_v, cand3_v,
                  candi0_v, candi1_v, candi2_v, candi3_v):
    wid = lax.axis_index("s") * NC + lax.axis_index("c")
    pltpu.sync_copy(pre_hbm.at[wid], row_v)

    neg = jnp.float32(_NEG)

    def boot(i, c):
        ca, cb = c
        base = i * (8 * L)
        for u in range(0, 8, 2):
            ca = jnp.maximum(ca, row_v[pl.ds(base + u * L, L)])
            cb = jnp.maximum(cb, row_v[pl.ds(base + (u + 1) * L, L)])
        return ca, cb

    ca, cb = lax.fori_loop(0, NVA // 8, boot,
                           (jnp.full((L,), neg), jnp.full((L,), neg)))
    t0 = lax.reduce_min(jnp.minimum(ca, cb), (0,))

    init_top = (jnp.full((L,), neg), jnp.zeros((L,), jnp.int32),
                jnp.full((L,), neg), jnp.zeros((L,), jnp.int32))
    hik, hii, lok, loi = _compact_topk(
        row_v, NVA, [cand0_v, cand1_v, cand2_v, cand3_v],
        [candi0_v, candi1_v, candi2_v, candi3_v], t0, 0, init_top)

    stv_v[pl.ds(0, L)] = hik
    stv_v[pl.ds(L, L)] = lok
    sti_v[pl.ds(0, L)] = hii
    sti_v[pl.ds(L, L)] = loi
    pltpu.sync_copy(stv_v, vout_hbm.at[wid])
    pltpu.sync_copy(sti_v, iout_hbm.at[wid])


def _sc_scan(pre1):
    mesh = plsc.VectorSubcoreMesh(
        core_axis_name="c", subcore_axis_name="s",
        num_cores=NC, num_subcores=NS)
    fn = functools.partial(
        pl.kernel,
        out_type=(jax.ShapeDtypeStruct((BT, 2 * L), jnp.float32),
                  jax.ShapeDtypeStruct((BT, 2 * L), jnp.int32)),
        mesh=mesh,
        scratch_types=[
            pltpu.VMEM((NFA,), jnp.float32),
            pltpu.VMEM((2 * L,), jnp.float32),
            pltpu.VMEM((2 * L,), jnp.int32),
        ] + _CAND_SCRATCH,
        compiler_params=pltpu.CompilerParams(needs_layout_passes=False),
        name="sc_scan",
    )(_sc_scan_body)
    return fn(pre1)


def _sc_final_body(pre_hbm, v1_hbm, i1_hbm, wdec_hbm, bdec_hbm,
                   f_hbm, recon_hbm,
                   row_v, fst_v, bufa_v, bufb_v, acc_v, bdec_v, idx_v,
                   stv_v, sti_v,
                   cand0_v, cand1_v, cand2_v, cand3_v,
                   candi0_v, candi1_v, candi2_v, candi3_v,
                   semg_a, semg_b, semb, semf, semr):
    wid = lax.axis_index("s") * NC + lax.axis_index("c")
    cpb = pltpu.async_copy(bdec_hbm, bdec_v, semb)
    cpr = pltpu.async_copy(pre_hbm.at[wid], row_v, semr)
    pltpu.sync_copy(v1_hbm.at[wid], stv_v)
    pltpu.sync_copy(i1_hbm.at[wid], sti_v)

    zero = jnp.zeros((L,), jnp.float32)

    def zbody(i, _):
        base = i * (8 * L)
        for u in range(8):
            fst_v[pl.ds(base + u * L, L)] = zero
        return 0

    lax.fori_loop(0, NF // (8 * L), zbody, 0)

    hik = stv_v[pl.ds(0, L)]
    lok = stv_v[pl.ds(L, L)]
    hii = sti_v[pl.ds(0, L)]
    loi = sti_v[pl.ds(L, L)]
    t0 = lax.reduce_min(lok, (0,))
    cpr.wait()

    hik, hii, lok, loi = _compact_topk(
        row_v, NVB, [cand0_v, cand1_v, cand2_v, cand3_v],
        [candi0_v, candi1_v, candi2_v, candi3_v], t0, NFA,
        (hik, hii, lok, loi))

    idx_v[pl.ds(0, L)] = hii
    idx_v[pl.ds(L, L)] = loi
    gathers = [
        pltpu.async_copy(
            wdec_hbm.at[idx_v.at[pl.ds(0, GROWS)]], bufa_v, semg_a)
    ]

    plsc.store_scatter(fst_v, [hii], jnp.maximum(hik, 0.0))
    plsc.store_scatter(fst_v, [loi], jnp.maximum(lok, 0.0))
    cpf = pltpu.async_copy(fst_v, f_hbm.at[wid], semf)

    vh = jnp.maximum(hik, 0.0)
    vl = jnp.maximum(lok, 0.0)
    ws_all = [vh[r] for r in range(L)] + [vl[r] for r in range(L)]
    cpb.wait()

    nchunks = (2 * L) // GROWS
    for k in range(nchunks):
        if k + 1 < nchunks:
            gathers.append(pltpu.async_copy(
                wdec_hbm.at[idx_v.at[pl.ds((k + 1) * GROWS, GROWS)]],
                bufb_v if k % 2 == 0 else bufa_v,
                semg_b if k % 2 == 0 else semg_a))
        gathers[k].wait()
        buf = bufa_v if k % 2 == 0 else bufb_v
        ws = ws_all[k * GROWS:(k + 1) * GROWS]
        src = bdec_v if k == 0 else acc_v

        def jbody(j, _, buf=buf, ws=ws, src=src):
            for q in range(4):
                o = j * (4 * L) + q * L
                t = [ws[r] * buf[r, pl.ds(o, L)] for r in range(GROWS)]
                t = [t[2 * i] + t[2 * i + 1] for i in range(GROWS // 2)]
                t = [t[2 * i] + t[2 * i + 1] for i in range(GROWS // 4)]
                acc_v[pl.ds(o, L)] = src[pl.ds(o, L)] + t[0] + t[1]
            return 0

        lax.fori_loop(0, DM // (4 * L), jbody, 0)

    cpf.wait()
    pltpu.sync_copy(acc_v, recon_hbm.at[wid])


def _sc_final(pre2, v1, i1, W_dec, b_dec):
    mesh = plsc.VectorSubcoreMesh(
        core_axis_name="c", subcore_axis_name="s",
        num_cores=NC, num_subcores=NS)
    fn = functools.partial(
        pl.kernel,
        out_type=(jax.ShapeDtypeStruct((BT, NF), jnp.float32),
                  jax.ShapeDtypeStruct((BT, DM), jnp.float32)),
        mesh=mesh,
        scratch_types=[
            pltpu.VMEM((NFB,), jnp.float32),
            pltpu.VMEM((NF,), jnp.float32),
            pltpu.VMEM((GROWS, DM), jnp.float32),
            pltpu.VMEM((GROWS, DM), jnp.float32),
            pltpu.VMEM((DM,), jnp.float32),
            pltpu.VMEM((DM,), jnp.float32),
            pltpu.VMEM((2 * L,), jnp.int32),
            pltpu.VMEM((2 * L,), jnp.float32),
            pltpu.VMEM((2 * L,), jnp.int32),
        ] + _CAND_SCRATCH + [
            pltpu.SemaphoreType.DMA,
            pltpu.SemaphoreType.DMA,
            pltpu.SemaphoreType.DMA,
            pltpu.SemaphoreType.DMA,
            pltpu.SemaphoreType.DMA,
        ],
        compiler_params=pltpu.CompilerParams(needs_layout_passes=False),
        name="sc_final",
    )(_sc_final_body)
    return fn(pre2, v1, i1, W_dec, b_dec)


def kernel(x, W_enc, b_enc, W_dec, b_dec):
    pre1 = _encode_part(x, W_enc, b_enc, b_dec, 0, NFA, "a")
    v1, i1 = _sc_scan(pre1)
    pre2 = _encode_part(x, W_enc, b_enc, b_dec, NFA, NFB, "b")
    f, recon = _sc_final(pre2, v1, i1, W_dec, b_dec)
    return (recon, f)
